# baseline (speedup 1.0000x reference)
import functools

import jax
import jax.numpy as jnp
from jax.experimental import pallas as pl
from jax.experimental.pallas import tpu as pltpu

_EPS = 1e-12


def _chamfer_body(a2r_ref, a1t_ref, n1_ref, n1t_ref, n2t_ref, n2r_ref,
                  d1_ref, nd1_ref, d2_ref, nd2_ref,
                  accd_ref, accn_ref):
    mt = pl.program_id(1)
    nmt = pl.num_programs(1)

    a2r = a2r_ref[0]
    a1t = a1t_ref[0]
    n1 = n1_ref[0]
    n1t = n1t_ref[0]
    n2t = n2t_ref[0]
    n2r = n2r_ref[0]

    dT = jnp.dot(a2r, a1t, preferred_element_type=jnp.float32)

    rmin = jnp.min(dT, axis=0, keepdims=True)
    oh1 = (dT == rmin).astype(jnp.bfloat16)
    cand_n = jnp.dot(n2t, oh1, preferred_element_type=jnp.float32)

    @pl.when(mt == 0)
    def _():
        accd_ref[...] = rmin
        accn_ref[...] = cand_n

    @pl.when(mt > 0)
    def _():
        prev = accd_ref[...]
        upd = rmin < prev
        accd_ref[...] = jnp.where(upd, rmin, prev)
        accn_ref[...] = jnp.where(upd, cand_n, accn_ref[...])

    cmin = jnp.min(dT, axis=1, keepdims=True)
    oh2 = (dT == cmin).astype(jnp.bfloat16)
    tn2 = jnp.dot(oh2, n1, preferred_element_type=jnp.float32)

    d2_ref[0] = cmin

    invt2 = 1.0 / jnp.maximum(
        jnp.sqrt(jnp.sum(tn2 * tn2, axis=1, keepdims=True)), _EPS)
    diff2 = n2r - tn2 * invt2
    nd2_ref[0] = jnp.sum(diff2 * diff2, axis=1, keepdims=True)

    @pl.when(mt == nmt - 1)
    def _():
        an = accn_ref[...]
        d1_ref[0] = accd_ref[...]
        invt1 = 1.0 / jnp.maximum(
            jnp.sqrt(jnp.sum(an * an, axis=0, keepdims=True)), _EPS)
        diff = n1t - an * invt1
        nd1_ref[0] = jnp.sum(diff * diff, axis=0, keepdims=True)


@functools.partial(jax.jit, static_argnames=("tm",))
def _chamfer(xyz1, xyz2, tm=1024):
    B, N, _ = xyz1.shape
    M = xyz2.shape[1]
    f32 = jnp.float32

    p1 = xyz1[:, :, :3]
    n1 = xyz1[:, :, 3:]
    p2 = xyz2[:, :, :3]
    n2 = xyz2[:, :, 3:]
    n1 = n1 / jnp.maximum(
        jnp.sqrt(jnp.sum(n1 * n1, axis=2, keepdims=True)), _EPS)
    n2 = n2 / jnp.maximum(
        jnp.sqrt(jnp.sum(n2 * n2, axis=2, keepdims=True)), _EPS)
    sq1 = jnp.sum(p1 * p1, axis=2, keepdims=True)
    sq2 = jnp.sum(p2 * p2, axis=2, keepdims=True)
    a1 = jnp.concatenate([p1, sq1, jnp.ones((B, N, 1), f32),
                          jnp.zeros((B, N, 3), f32)], axis=2)
    a2 = jnp.concatenate([-2.0 * p2, jnp.ones((B, M, 1), f32), sq2,
                          jnp.zeros((B, M, 3), f32)], axis=2)
    a1t = jnp.transpose(a1, (0, 2, 1))
    n1t = jnp.transpose(n1, (0, 2, 1))
    n1b = n1.astype(jnp.bfloat16)
    n2t = jnp.transpose(n2, (0, 2, 1)).astype(jnp.bfloat16)

    grid = (B, M // tm)
    d1, nd1, d2, nd2 = pl.pallas_call(
        _chamfer_body,
        grid=grid,
        in_specs=[
            pl.BlockSpec((1, tm, 8), lambda b, m: (b, m, 0)),
            pl.BlockSpec((1, 8, N), lambda b, m: (b, 0, 0)),
            pl.BlockSpec((1, N, 3), lambda b, m: (b, 0, 0)),
            pl.BlockSpec((1, 3, N), lambda b, m: (b, 0, 0)),
            pl.BlockSpec((1, 3, tm), lambda b, m: (b, 0, m)),
            pl.BlockSpec((1, tm, 3), lambda b, m: (b, m, 0)),
        ],
        out_specs=[
            pl.BlockSpec((1, 1, N), lambda b, m: (b, 0, 0)),
            pl.BlockSpec((1, 1, N), lambda b, m: (b, 0, 0)),
            pl.BlockSpec((1, tm, 1), lambda b, m: (b, m, 0)),
            pl.BlockSpec((1, tm, 1), lambda b, m: (b, m, 0)),
        ],
        out_shape=[
            jax.ShapeDtypeStruct((B, 1, N), f32),
            jax.ShapeDtypeStruct((B, 1, N), f32),
            jax.ShapeDtypeStruct((B, M, 1), f32),
            jax.ShapeDtypeStruct((B, M, 1), f32),
        ],
        scratch_shapes=[
            pltpu.VMEM((1, N), f32),
            pltpu.VMEM((3, N), f32),
        ],
        compiler_params=pltpu.CompilerParams(
            dimension_semantics=("parallel", "arbitrary")),
    )(a2, a1t, n1b, n1t, n2t, n2)
    return jnp.mean(d1) + jnp.mean(d2) + jnp.mean(nd1) + jnp.mean(nd2)


def kernel(xyz1, xyz2):
    return _chamfer(xyz1, xyz2)


# speedup vs baseline: 1.0047x; 1.0047x over previous
import functools

import jax
import jax.numpy as jnp
from jax.experimental import pallas as pl
from jax.experimental.pallas import tpu as pltpu

_EPS = 1e-12


def _chamfer_body(a2r_ref, a1t_ref, n1_ref, n1t_ref, n2t_ref, n2r_ref,
                  d1_ref, nd1_ref, d2_ref, nd2_ref,
                  accd_ref, accn_ref):
    mt = pl.program_id(1)
    nmt = pl.num_programs(1)

    a2r = a2r_ref[0]
    a1t = a1t_ref[0]
    n1 = n1_ref[0]
    n1t = n1t_ref[0]
    n2t = n2t_ref[0]
    n2r = n2r_ref[0]

    dT = jnp.dot(a2r, a1t, preferred_element_type=jnp.float32)

    rmin = jnp.min(dT, axis=0, keepdims=True)
    oh1 = (dT == rmin).astype(jnp.bfloat16)
    cand_n = jnp.dot(n2t, oh1, preferred_element_type=jnp.float32)

    @pl.when(mt == 0)
    def _():
        accd_ref[...] = rmin
        accn_ref[...] = cand_n

    @pl.when(mt > 0)
    def _():
        prev = accd_ref[...]
        upd = rmin < prev
        accd_ref[...] = jnp.where(upd, rmin, prev)
        accn_ref[...] = jnp.where(upd, cand_n, accn_ref[...])

    cmin = jnp.min(dT, axis=1, keepdims=True)
    oh2 = (dT == cmin).astype(jnp.bfloat16)
    tn2 = jnp.dot(oh2, n1, preferred_element_type=jnp.float32)

    d2_ref[0] = cmin

    invt2 = 1.0 / jnp.maximum(
        jnp.sqrt(jnp.sum(tn2 * tn2, axis=1, keepdims=True)), _EPS)
    diff2 = n2r - tn2 * invt2
    nd2_ref[0] = jnp.sum(diff2 * diff2, axis=1, keepdims=True)

    @pl.when(mt == nmt - 1)
    def _():
        an = accn_ref[...]
        d1_ref[0] = accd_ref[...]
        invt1 = 1.0 / jnp.maximum(
            jnp.sqrt(jnp.sum(an * an, axis=0, keepdims=True)), _EPS)
        diff = n1t - an * invt1
        nd1_ref[0] = jnp.sum(diff * diff, axis=0, keepdims=True)


@functools.partial(jax.jit, static_argnames=("tm",))
def _chamfer(xyz1, xyz2, tm=1024):
    B, N, _ = xyz1.shape
    M = xyz2.shape[1]
    f32 = jnp.float32

    p1 = xyz1[:, :, :3]
    n1 = xyz1[:, :, 3:]
    p2 = xyz2[:, :, :3]
    n2 = xyz2[:, :, 3:]
    n1 = n1 / jnp.maximum(
        jnp.sqrt(jnp.sum(n1 * n1, axis=2, keepdims=True)), _EPS)
    n2 = n2 / jnp.maximum(
        jnp.sqrt(jnp.sum(n2 * n2, axis=2, keepdims=True)), _EPS)
    sq1 = jnp.sum(p1 * p1, axis=2, keepdims=True)
    sq2 = jnp.sum(p2 * p2, axis=2, keepdims=True)
    a1 = jnp.concatenate([p1, sq1, jnp.ones((B, N, 1), f32),
                          jnp.zeros((B, N, 3), f32)], axis=2)
    a2 = jnp.concatenate([-2.0 * p2, jnp.ones((B, M, 1), f32), sq2,
                          jnp.zeros((B, M, 3), f32)], axis=2)
    a1t = jnp.transpose(a1, (0, 2, 1))
    n1t = jnp.transpose(n1, (0, 2, 1))
    n1b = n1.astype(jnp.bfloat16)
    n2t = jnp.transpose(n2, (0, 2, 1)).astype(jnp.bfloat16)

    grid = (B, M // tm)
    d1, nd1, d2, nd2 = pl.pallas_call(
        _chamfer_body,
        grid=grid,
        in_specs=[
            pl.BlockSpec((1, tm, 8), lambda b, m: (b, m, 0)),
            pl.BlockSpec((1, 8, N), lambda b, m: (b, 0, 0)),
            pl.BlockSpec((1, N, 3), lambda b, m: (b, 0, 0)),
            pl.BlockSpec((1, 3, N), lambda b, m: (b, 0, 0)),
            pl.BlockSpec((1, 3, tm), lambda b, m: (b, 0, m)),
            pl.BlockSpec((1, tm, 3), lambda b, m: (b, m, 0)),
        ],
        out_specs=[
            pl.BlockSpec((1, 1, N), lambda b, m: (b, 0, 0)),
            pl.BlockSpec((1, 1, N), lambda b, m: (b, 0, 0)),
            pl.BlockSpec((1, tm, 1), lambda b, m: (b, m, 0)),
            pl.BlockSpec((1, tm, 1), lambda b, m: (b, m, 0)),
        ],
        out_shape=[
            jax.ShapeDtypeStruct((B, 1, N), f32),
            jax.ShapeDtypeStruct((B, 1, N), f32),
            jax.ShapeDtypeStruct((B, M, 1), f32),
            jax.ShapeDtypeStruct((B, M, 1), f32),
        ],
        scratch_shapes=[
            pltpu.VMEM((1, N), f32),
            pltpu.VMEM((3, N), f32),
        ],
    )(a2, a1t, n1b, n1t, n2t, n2)
    return jnp.mean(d1) + jnp.mean(d2) + jnp.mean(nd1) + jnp.mean(nd2)


def kernel(xyz1, xyz2):
    return _chamfer(xyz1, xyz2)


# speedup vs baseline: 1.4034x; 1.3968x over previous
import functools

import jax
import jax.numpy as jnp
from jax.experimental import pallas as pl
from jax.experimental.pallas import tpu as pltpu

_EPS = 1e-12


def _make_body(n_chunks, nc):
    def _chamfer_body(x2_ref, a1t_ref, n1_ref, n1t_ref, n2t_ref, n2r_ref,
                      out_ref):
        x2 = x2_ref[0]
        n1 = n1_ref[0]
        n2t = n2t_ref[0]
        n2r = n2r_ref[0]

        p2 = x2[:, 0:3]
        sq2 = jnp.sum(p2 * p2, axis=1, keepdims=True)
        one = jnp.ones_like(sq2)
        a2r = jnp.concatenate(
            [-2.0 * p2, one, sq2, jnp.zeros_like(p2)], axis=1)

        s_d1 = 0.0
        s_nd1 = 0.0
        cmin_acc = None
        tn2_acc = None
        for c in range(n_chunks):
            a1c = a1t_ref[0, :, pl.ds(c * nc, nc)]
            dc = jnp.dot(a2r, a1c, preferred_element_type=jnp.float32)

            rmin = jnp.min(dc, axis=0, keepdims=True)
            oh1 = (dc == rmin).astype(jnp.bfloat16)
            cand = jnp.dot(n2t, oh1, preferred_element_type=jnp.float32)
            s_d1 = s_d1 + jnp.sum(rmin)

            invt1 = 1.0 / jnp.maximum(
                jnp.sqrt(jnp.sum(cand * cand, axis=0, keepdims=True)), _EPS)
            n1tc = n1t_ref[0, :, pl.ds(c * nc, nc)]
            diff1 = n1tc - cand * invt1
            s_nd1 = s_nd1 + jnp.sum(diff1 * diff1)

            cmin = jnp.min(dc, axis=1, keepdims=True)
            oh2 = (dc == cmin).astype(jnp.bfloat16)
            n1c = n1_ref[0, pl.ds(c * nc, nc), :]
            tn2 = jnp.dot(oh2, n1c, preferred_element_type=jnp.float32)
            if c == 0:
                cmin_acc, tn2_acc = cmin, tn2
            else:
                upd = cmin < cmin_acc
                cmin_acc = jnp.where(upd, cmin, cmin_acc)
                tn2_acc = jnp.where(upd, tn2, tn2_acc)

        s_d2 = jnp.sum(cmin_acc)
        invt2 = 1.0 / jnp.maximum(
            jnp.sqrt(jnp.sum(tn2_acc * tn2_acc, axis=1, keepdims=True)), _EPS)
        diff2 = n2r - tn2_acc * invt2
        s_nd2 = jnp.sum(diff2 * diff2)

        out_ref[0, 0] = jnp.full((128,), s_d1, jnp.float32)
        out_ref[0, 1] = jnp.full((128,), s_nd1, jnp.float32)
        out_ref[0, 2] = jnp.full((128,), s_d2, jnp.float32)
        out_ref[0, 3] = jnp.full((128,), s_nd2, jnp.float32)

    return _chamfer_body


@functools.partial(jax.jit, static_argnames=("nc",))
def _chamfer(xyz1, xyz2, nc=1024):
    B, N, _ = xyz1.shape
    M = xyz2.shape[1]
    f32 = jnp.float32

    p1 = xyz1[:, :, :3]
    n1 = xyz1[:, :, 3:]
    n2 = xyz2[:, :, 3:]
    n1 = n1 / jnp.maximum(
        jnp.sqrt(jnp.sum(n1 * n1, axis=2, keepdims=True)), _EPS)
    n2 = n2 / jnp.maximum(
        jnp.sqrt(jnp.sum(n2 * n2, axis=2, keepdims=True)), _EPS)
    sq1 = jnp.sum(p1 * p1, axis=2, keepdims=True)
    a1 = jnp.concatenate([p1, sq1, jnp.ones((B, N, 1), f32),
                          jnp.zeros((B, N, 3), f32)], axis=2)
    a1t = jnp.transpose(a1, (0, 2, 1))
    n1t = jnp.transpose(n1, (0, 2, 1))
    n1b = n1.astype(jnp.bfloat16)
    n2t = jnp.transpose(n2, (0, 2, 1)).astype(jnp.bfloat16)

    sums = pl.pallas_call(
        _make_body(N // nc, nc),
        grid=(B,),
        in_specs=[
            pl.BlockSpec((1, M, 6), lambda b: (b, 0, 0)),
            pl.BlockSpec((1, 8, N), lambda b: (b, 0, 0)),
            pl.BlockSpec((1, N, 3), lambda b: (b, 0, 0)),
            pl.BlockSpec((1, 3, N), lambda b: (b, 0, 0)),
            pl.BlockSpec((1, 3, M), lambda b: (b, 0, 0)),
            pl.BlockSpec((1, M, 3), lambda b: (b, 0, 0)),
        ],
        out_specs=pl.BlockSpec((1, 4, 128), lambda b: (b, 0, 0)),
        out_shape=jax.ShapeDtypeStruct((B, 4, 128), f32),
    )(xyz2, a1t, n1b, n1t, n2t, n2)
    s = jnp.sum(sums[:, :, 0], axis=0)
    return (s[0] + s[1]) / (B * N) + (s[2] + s[3]) / (B * M)


def kernel(xyz1, xyz2):
    return _chamfer(xyz1, xyz2)
